# Initial kernel scaffold; baseline (speedup 1.0000x reference)
import jax
import jax.numpy as jnp
from jax.experimental import pallas as pl
from jax.experimental.pallas import tpu as pltpu

_K = 8192
_D = 256
_BN = 512
_BK = 1024
_NKC = _K // _BK


def _argmin_body(z_ref, cb_ref, idx_ref):
    z = z_ref[...]
    zsq = jnp.sum(z * z, axis=1, keepdims=True)
    minval = jnp.full((_BN,), jnp.inf, dtype=jnp.float32)
    minidx = jnp.zeros((_BN,), dtype=jnp.int32)
    for j in range(_NKC):
        cb = cb_ref[pl.ds(j * _BK, _BK), :]
        cbsq = jnp.sum(cb * cb, axis=1)[None, :]
        cross = jax.lax.dot_general(
            z, cb, (((1,), (1,)), ((), ())),
            preferred_element_type=jnp.float32)
        d2 = jnp.maximum(zsq + cbsq - 2.0 * cross, 0.0)
        dist = jnp.sqrt(d2)
        lmin = jnp.min(dist, axis=1)
        kidx = jax.lax.broadcasted_iota(jnp.int32, (_BN, _BK), 1) + j * _BK
        lidx = jnp.min(
            jnp.where(dist == lmin[:, None], kidx, jnp.int32(2**30)), axis=1)
        upd = lmin < minval
        minval = jnp.where(upd, lmin, minval)
        minidx = jnp.where(upd, lidx, minidx)
    idx_ref[...] = minidx


def _nearest_indices(z_flat, codebook, interpret=False):
    n = z_flat.shape[0]
    grid = (n // _BN,)
    return pl.pallas_call(
        _argmin_body,
        grid=grid,
        in_specs=[
            pl.BlockSpec((_BN, _D), lambda i: (i, 0)),
            pl.BlockSpec((_K, _D), lambda i: (0, 0)),
        ],
        out_specs=pl.BlockSpec((_BN,), lambda i: (i,)),
        out_shape=jax.ShapeDtypeStruct((n,), jnp.int32),
        compiler_params=pltpu.CompilerParams(
            dimension_semantics=("arbitrary",),
        ),
        interpret=interpret,
    )(z_flat, codebook)


def kernel(z, codebook):
    z_flat = z.reshape(-1, _D)
    idx = _nearest_indices(z_flat, codebook)
    z_q = jnp.take(codebook, idx, axis=0)
    return z_q.reshape(z.shape)


# speedup vs baseline: 1.0088x; 1.0088x over previous
import functools

import jax
import jax.numpy as jnp
from jax import lax
from jax.experimental import pallas as pl
from jax.experimental.pallas import tpu as pltpu
from jax.experimental.pallas import tpu_sc as plsc

_K = 8192
_D = 256
_BN = 512
_BK = 1024
_NKC = _K // _BK


def _argmin_body(z_ref, cb_ref, zsq_ref, cbsq_ref, idx_ref):
    z = z_ref[...]
    zsq = zsq_ref[...].reshape(_BN, 1)
    cbsq_row = cbsq_ref[...]
    minval = jnp.full((_BN,), jnp.inf, dtype=jnp.float32)
    minidx = jnp.zeros((_BN,), dtype=jnp.int32)
    for j in range(_NKC):
        cb = cb_ref[pl.ds(j * _BK, _BK), :]
        cbsq = cbsq_row[:, j * _BK:(j + 1) * _BK]
        cross = jax.lax.dot_general(
            z, cb, (((1,), (1,)), ((), ())),
            preferred_element_type=jnp.float32)
        d2 = jnp.maximum(zsq + cbsq - 2.0 * cross, 0.0)
        dist = jnp.sqrt(d2)
        lmin = jnp.min(dist, axis=1)
        kidx = jax.lax.broadcasted_iota(jnp.int32, (_BN, _BK), 1) + j * _BK
        lidx = jnp.min(
            jnp.where(dist == lmin[:, None], kidx, jnp.int32(2**30)), axis=1)
        upd = lmin < minval
        minval = jnp.where(upd, lmin, minval)
        minidx = jnp.where(upd, lidx, minidx)
    idx_ref[...] = minidx.reshape(1, 1, _BN)


def _nearest_indices(z_flat, codebook, zsq, cbsq, interpret=False):
    n = z_flat.shape[0]
    nb = n // _BN
    return pl.pallas_call(
        _argmin_body,
        grid=(nb,),
        in_specs=[
            pl.BlockSpec((_BN, _D), lambda i: (i, 0)),
            pl.BlockSpec((_K, _D), lambda i: (0, 0)),
            pl.BlockSpec((1, 1, _BN), lambda i: (i, 0, 0)),
            pl.BlockSpec((1, _K), lambda i: (0, 0)),
        ],
        out_specs=pl.BlockSpec((1, 1, _BN), lambda i: (i, 0, 0)),
        out_shape=jax.ShapeDtypeStruct((nb, 1, _BN), jnp.int32),
        compiler_params=pltpu.CompilerParams(
            dimension_semantics=("arbitrary",),
        ),
        interpret=interpret,
    )(z_flat, codebook, zsq.reshape(nb, 1, _BN), cbsq)


_SC_CHUNK = 128


def _sc_gather(table, idx):
    b = idx.shape[0]
    d = table.shape[1]
    info = plsc.get_sparse_core_info()
    nw = info.num_cores * info.num_subcores
    b_per_w = b // nw
    nchunks = b_per_w // _SC_CHUNK
    mesh = plsc.VectorSubcoreMesh(core_axis_name="c", subcore_axis_name="s")

    @functools.partial(
        pl.kernel, mesh=mesh,
        out_type=jax.ShapeDtypeStruct((b, d), jnp.float32),
        scratch_types=[
            pltpu.VMEM((b_per_w,), jnp.int32),
            pltpu.VMEM((_SC_CHUNK, d), jnp.float32),
            pltpu.SemaphoreType.DMA,
        ],
    )
    def k(table_hbm, idx_hbm, out_hbm, idx_v, rows_v, sem):
        wid = lax.axis_index("s") * info.num_cores + lax.axis_index("c")
        base = wid * b_per_w
        pltpu.sync_copy(idx_hbm.at[pl.ds(base, b_per_w)], idx_v)
        for c in range(nchunks):
            pltpu.async_copy(
                table_hbm.at[idx_v.at[pl.ds(c * _SC_CHUNK, _SC_CHUNK)]],
                rows_v, sem).wait()
            pltpu.sync_copy(
                rows_v, out_hbm.at[pl.ds(base + c * _SC_CHUNK, _SC_CHUNK)])

    return k(table, idx)


def kernel(z, codebook):
    z_flat = z.reshape(-1, _D)
    zsq = jnp.sum(z_flat * z_flat, axis=1, keepdims=True)
    cbsq = jnp.sum(codebook * codebook, axis=1)[None, :]
    idx = _nearest_indices(z_flat, codebook, zsq, cbsq).reshape(-1)
    z_q = _sc_gather(codebook, idx)
    return z_q.reshape(z.shape)
